# baseline (speedup 1.0000x reference)
import jax, jax.numpy as jnp
import numpy as np
from jax.experimental import pallas as pl


def _index_points(points, idx):
    B = points.shape[0]
    bidx = jnp.arange(B).reshape((B,) + (1,) * (idx.ndim - 1))
    return points[bidx, idx]


def _square_distance(src, dst):
    d = -2.0 * jnp.matmul(src, dst.transpose(0, 2, 1))
    d = d + jnp.sum(src ** 2, -1)[:, :, None]
    d = d + jnp.sum(dst ** 2, -1)[:, None, :]
    return d


def _fps(xyz, npoint):
    xyz = jax.lax.stop_gradient(xyz)
    B, N, _ = xyz.shape
    def body(carry, _):
        distance, farthest = carry
        centroid = xyz[jnp.arange(B), farthest][:, None, :]
        dist = jnp.sum((xyz - centroid) ** 2, -1)
        distance = jnp.minimum(distance, dist)
        new_far = jnp.argmax(distance, -1).astype(jnp.int32)
        return (distance, new_far), farthest
    init = (jnp.full((B, N), 1e10, jnp.float32), jnp.zeros((B,), jnp.int32))
    _, idx = jax.lax.scan(body, init, None, length=npoint)
    return jnp.transpose(idx)


def _query_ball(radius, nsample, xyz, new_xyz):
    B, N, _ = xyz.shape
    S = new_xyz.shape[1]
    sqrdists = jax.lax.stop_gradient(_square_distance(new_xyz, xyz))
    group_idx = jnp.broadcast_to(jnp.arange(N, dtype=jnp.int32), (B, S, N))
    group_idx = jnp.where(sqrdists > radius ** 2, N, group_idx)
    group_idx = jnp.sort(group_idx, axis=-1)[:, :, :nsample]
    group_first = group_idx[:, :, :1]
    group_idx = jnp.where(group_idx == N, group_first, group_idx)
    return group_idx


def _mlp_apply(x, layers):
    for p in layers:
        sh = (1, -1) + (1,) * (x.ndim - 2)
        y = jnp.einsum('oi,bi...->bo...', p['w'], x) + p['b'].reshape(sh)
        axes = (0,) + tuple(range(2, y.ndim))
        mean = jnp.mean(y, axes, keepdims=True)
        var = jnp.var(y, axes, keepdims=True)
        y = (y - mean) / jnp.sqrt(var + 1e-5)
        y = y * p['g'].reshape(sh) + p['be'].reshape(sh)
        x = jax.nn.relu(y)
    return x


def _sa_msg(xyz, points, npoint, radius_list, nsample_list, branch_params):
    xyz_t = xyz.transpose(0, 2, 1)
    points_t = points.transpose(0, 2, 1)
    fps_idx = _fps(xyz_t, npoint)
    new_xyz = _index_points(xyz_t, fps_idx)
    outs = []
    for radius, K, layers in zip(radius_list, nsample_list, branch_params):
        gidx = _query_ball(radius, K, xyz_t, new_xyz)
        grouped_xyz = _index_points(xyz_t, gidx) - new_xyz[:, :, None, :]
        grouped_points = jnp.concatenate([_index_points(points_t, gidx), grouped_xyz], -1)
        g = _mlp_apply(grouped_points.transpose(0, 3, 2, 1), layers)
        outs.append(jnp.max(g, axis=2))
    return new_xyz.transpose(0, 2, 1), jnp.concatenate(outs, axis=1)


def _sa_all(xyz, points, layers):
    xyz_t = xyz.transpose(0, 2, 1)
    points_t = points.transpose(0, 2, 1)
    new_points = jnp.concatenate([xyz_t[:, None], points_t[:, None]], -1)
    g = _mlp_apply(new_points.transpose(0, 3, 2, 1), layers)
    return jnp.zeros((xyz.shape[0], 3, 1), xyz.dtype), jnp.max(g, axis=2)


def _fp(xyz1, xyz2, points1, points2, layers):
    xyz1_t = xyz1.transpose(0, 2, 1)
    xyz2_t = xyz2.transpose(0, 2, 1)
    points2_t = points2.transpose(0, 2, 1)
    B, N, _ = xyz1_t.shape
    S = xyz2_t.shape[1]
    if S == 1:
        interpolated = jnp.tile(points2_t, (1, N, 1))
    else:
        dists = _square_distance(xyz1_t, xyz2_t)
        neg, idx = jax.lax.top_k(-dists, 3)
        d3 = -neg
        recip = 1.0 / (d3 + 1e-8)
        weight = recip / jnp.sum(recip, 2, keepdims=True)
        interpolated = jnp.sum(_index_points(points2_t, idx) * weight[..., None], axis=2)
    if points1 is not None:
        new_points = jnp.concatenate([points1.transpose(0, 2, 1), interpolated], -1)
    else:
        new_points = interpolated
    return _mlp_apply(new_points.transpose(0, 2, 1), layers)


def _pallas_identity(x):
    def body(x_ref, o_ref):
        o_ref[...] = x_ref[...]
    return pl.pallas_call(
        body, out_shape=jax.ShapeDtypeStruct(x.shape, x.dtype))(x)


def kernel(xyz, cls_label, params):
    B, C, N = xyz.shape
    l0_xyz = xyz[:, :3, :]
    l1_xyz, l1_points = _sa_msg(l0_xyz, xyz, 512, [0.1, 0.2, 0.4], [32, 64, 128], params['sa1'])
    l2_xyz, l2_points = _sa_msg(l1_xyz, l1_points, 128, [0.4, 0.8], [64, 128], params['sa2'])
    l3_xyz, l3_points = _sa_all(l2_xyz, l2_points, params['sa3'])
    l2_up = _fp(l2_xyz, l3_xyz, l2_points, l3_points, params['fp3'])
    l1_up = _fp(l1_xyz, l2_xyz, l1_points, l2_up, params['fp2'])
    cls_oh = jnp.tile(cls_label.reshape(B, 16, 1), (1, 1, N))
    l0_points = _fp(l0_xyz, l1_xyz, jnp.concatenate([cls_oh, l0_xyz, xyz], axis=1), l1_up, params['fp1'])
    y = jnp.einsum('oi,bin->bon', params['conv1']['w'], l0_points) + params['conv1']['b'][None, :, None]
    mean = jnp.mean(y, (0, 2), keepdims=True)
    var = jnp.var(y, (0, 2), keepdims=True)
    y = (y - mean) / jnp.sqrt(var + 1e-5)
    y = y * params['bn1']['g'][None, :, None] + params['bn1']['be'][None, :, None]
    y = jax.nn.relu(y)
    y = jnp.einsum('oi,bin->bon', params['conv2']['w'], y) + params['conv2']['b'][None, :, None]
    y = jax.nn.log_softmax(y, axis=1)
    y = _pallas_identity(y)
    return (y.transpose(0, 2, 1), l3_points)


# speedup vs baseline: 1.0798x; 1.0798x over previous
import jax, jax.numpy as jnp
import numpy as np
from jax.experimental import pallas as pl


def _index_points(points, idx):
    B = points.shape[0]
    bidx = jnp.arange(B).reshape((B,) + (1,) * (idx.ndim - 1))
    return points[bidx, idx]


def _square_distance(src, dst):
    d = -2.0 * jnp.matmul(src, dst.transpose(0, 2, 1))
    d = d + jnp.sum(src ** 2, -1)[:, :, None]
    d = d + jnp.sum(dst ** 2, -1)[:, None, :]
    return d


def _fps(xyz_t, npoint, interpret=False):
    B, N, _ = xyz_t.shape
    x = xyz_t[:, :, 0]
    y = xyz_t[:, :, 1]
    z = xyz_t[:, :, 2]

    def body(x_ref, y_ref, z_ref, o_ref):
        xv = x_ref[...]
        yv = y_ref[...]
        zv = z_ref[...]
        iota_n = jax.lax.broadcasted_iota(jnp.int32, (B, N), 1)
        iota_p = jax.lax.broadcasted_iota(jnp.int32, (B, npoint), 1)

        dist0 = (xv - xv[:, :1]) ** 2 + (yv - yv[:, :1]) ** 2 + (zv - zv[:, :1]) ** 2
        f0 = jnp.min(jnp.where(dist0 < -1.0, iota_n, 0), axis=1,
                     keepdims=True).astype(jnp.int32)
        o_ref[...] = jnp.where(iota_p >= 0, 0, iota_p)

        def step(i, carry):
            distance, f = carry
            o_ref[...] = jnp.where(iota_p == i, f, o_ref[...])
            m = iota_n == f
            cx = jnp.sum(jnp.where(m, xv, 0.0), axis=1, keepdims=True)
            cy = jnp.sum(jnp.where(m, yv, 0.0), axis=1, keepdims=True)
            cz = jnp.sum(jnp.where(m, zv, 0.0), axis=1, keepdims=True)
            dist = (xv - cx) ** 2 + (yv - cy) ** 2 + (zv - cz) ** 2
            distance = jnp.minimum(distance, dist)
            mx = jnp.max(distance, axis=1, keepdims=True)
            f = jnp.min(jnp.where(distance == mx, iota_n, N), axis=1,
                        keepdims=True).astype(jnp.int32)
            return distance, f

        jax.lax.fori_loop(0, npoint, step, (dist0, f0))

    return pl.pallas_call(
        body,
        out_shape=jax.ShapeDtypeStruct((B, npoint), jnp.int32),
        interpret=interpret,
    )(x, y, z)


def _query_ball(radius, nsample, xyz, new_xyz):
    B, N, _ = xyz.shape
    S = new_xyz.shape[1]
    sqrdists = jax.lax.stop_gradient(_square_distance(new_xyz, xyz))
    group_idx = jnp.broadcast_to(jnp.arange(N, dtype=jnp.int32), (B, S, N))
    group_idx = jnp.where(sqrdists > radius ** 2, N, group_idx)
    group_idx = jnp.sort(group_idx, axis=-1)[:, :, :nsample]
    group_first = group_idx[:, :, :1]
    group_idx = jnp.where(group_idx == N, group_first, group_idx)
    return group_idx


def _mlp_apply(x, layers):
    for p in layers:
        sh = (1, -1) + (1,) * (x.ndim - 2)
        y = jnp.einsum('oi,bi...->bo...', p['w'], x) + p['b'].reshape(sh)
        axes = (0,) + tuple(range(2, y.ndim))
        mean = jnp.mean(y, axes, keepdims=True)
        var = jnp.var(y, axes, keepdims=True)
        y = (y - mean) / jnp.sqrt(var + 1e-5)
        y = y * p['g'].reshape(sh) + p['be'].reshape(sh)
        x = jax.nn.relu(y)
    return x


def _sa_msg(xyz, points, npoint, radius_list, nsample_list, branch_params):
    xyz_t = xyz.transpose(0, 2, 1)
    points_t = points.transpose(0, 2, 1)
    fps_idx = _fps(xyz_t, npoint)
    new_xyz = _index_points(xyz_t, fps_idx)
    outs = []
    for radius, K, layers in zip(radius_list, nsample_list, branch_params):
        gidx = _query_ball(radius, K, xyz_t, new_xyz)
        grouped_xyz = _index_points(xyz_t, gidx) - new_xyz[:, :, None, :]
        grouped_points = jnp.concatenate([_index_points(points_t, gidx), grouped_xyz], -1)
        g = _mlp_apply(grouped_points.transpose(0, 3, 2, 1), layers)
        outs.append(jnp.max(g, axis=2))
    return new_xyz.transpose(0, 2, 1), jnp.concatenate(outs, axis=1)


def _sa_all(xyz, points, layers):
    xyz_t = xyz.transpose(0, 2, 1)
    points_t = points.transpose(0, 2, 1)
    new_points = jnp.concatenate([xyz_t[:, None], points_t[:, None]], -1)
    g = _mlp_apply(new_points.transpose(0, 3, 2, 1), layers)
    return jnp.zeros((xyz.shape[0], 3, 1), xyz.dtype), jnp.max(g, axis=2)


def _fp(xyz1, xyz2, points1, points2, layers):
    xyz1_t = xyz1.transpose(0, 2, 1)
    xyz2_t = xyz2.transpose(0, 2, 1)
    points2_t = points2.transpose(0, 2, 1)
    B, N, _ = xyz1_t.shape
    S = xyz2_t.shape[1]
    if S == 1:
        interpolated = jnp.tile(points2_t, (1, N, 1))
    else:
        dists = _square_distance(xyz1_t, xyz2_t)
        neg, idx = jax.lax.top_k(-dists, 3)
        d3 = -neg
        recip = 1.0 / (d3 + 1e-8)
        weight = recip / jnp.sum(recip, 2, keepdims=True)
        interpolated = jnp.sum(_index_points(points2_t, idx) * weight[..., None], axis=2)
    if points1 is not None:
        new_points = jnp.concatenate([points1.transpose(0, 2, 1), interpolated], -1)
    else:
        new_points = interpolated
    return _mlp_apply(new_points.transpose(0, 2, 1), layers)


def _pallas_identity(x):
    def body(x_ref, o_ref):
        o_ref[...] = x_ref[...]
    return pl.pallas_call(
        body, out_shape=jax.ShapeDtypeStruct(x.shape, x.dtype))(x)


def kernel(xyz, cls_label, params):
    B, C, N = xyz.shape
    l0_xyz = xyz[:, :3, :]
    l1_xyz, l1_points = _sa_msg(l0_xyz, xyz, 512, [0.1, 0.2, 0.4], [32, 64, 128], params['sa1'])
    l2_xyz, l2_points = _sa_msg(l1_xyz, l1_points, 128, [0.4, 0.8], [64, 128], params['sa2'])
    l3_xyz, l3_points = _sa_all(l2_xyz, l2_points, params['sa3'])
    l2_up = _fp(l2_xyz, l3_xyz, l2_points, l3_points, params['fp3'])
    l1_up = _fp(l1_xyz, l2_xyz, l1_points, l2_up, params['fp2'])
    cls_oh = jnp.tile(cls_label.reshape(B, 16, 1), (1, 1, N))
    l0_points = _fp(l0_xyz, l1_xyz, jnp.concatenate([cls_oh, l0_xyz, xyz], axis=1), l1_up, params['fp1'])
    y = jnp.einsum('oi,bin->bon', params['conv1']['w'], l0_points) + params['conv1']['b'][None, :, None]
    mean = jnp.mean(y, (0, 2), keepdims=True)
    var = jnp.var(y, (0, 2), keepdims=True)
    y = (y - mean) / jnp.sqrt(var + 1e-5)
    y = y * params['bn1']['g'][None, :, None] + params['bn1']['be'][None, :, None]
    y = jax.nn.relu(y)
    y = jnp.einsum('oi,bin->bon', params['conv2']['w'], y) + params['conv2']['b'][None, :, None]
    y = jax.nn.log_softmax(y, axis=1)
    y = _pallas_identity(y)
    return (y.transpose(0, 2, 1), l3_points)


# speedup vs baseline: 4.3789x; 4.0553x over previous
import functools

import jax, jax.numpy as jnp
import numpy as np
from jax import lax
from jax.experimental import pallas as pl
from jax.experimental.pallas import tpu as pltpu
from jax.experimental.pallas import tpu_sc as plsc


def _sc_gather(table, idx):
    V, D = table.shape
    R = idx.shape[0]
    info = plsc.get_sparse_core_info()
    NW = info.num_cores * info.num_subcores
    assert R % NW == 0, (R, NW)
    per_w = R // NW
    CH = min(128, per_w)
    assert per_w % CH == 0 and CH % 8 == 0, (per_w, CH)
    n_chunks = per_w // CH
    mesh = plsc.VectorSubcoreMesh(core_axis_name="c", subcore_axis_name="s")

    @functools.partial(
        pl.kernel, mesh=mesh,
        out_type=jax.ShapeDtypeStruct((R, D), jnp.float32),
        scratch_types=[
            pltpu.VMEM((CH,), jnp.int32),
            pltpu.VMEM((CH, D), jnp.float32),
            pltpu.SemaphoreType.DMA,
        ],
    )
    def k(table_hbm, idx_hbm, out_hbm, idx_v, rows_v, sem):
        wid = lax.axis_index("s") * info.num_cores + lax.axis_index("c")
        base = wid * per_w

        def body(i, carry):
            off = base + i * CH
            pltpu.sync_copy(idx_hbm.at[pl.ds(off, CH)], idx_v)
            pltpu.async_copy(table_hbm.at[idx_v], rows_v, sem).wait()
            pltpu.sync_copy(rows_v, out_hbm.at[pl.ds(off, CH)])
            return carry

        lax.fori_loop(0, n_chunks, body, 0)

    return k(table, idx)


def _pad_channels(x, d_to):
    c = x.shape[-1]
    if c == d_to:
        return x
    return jnp.concatenate(
        [x, jnp.zeros(x.shape[:-1] + (d_to - c,), x.dtype)], axis=-1)


def _gather_points(table_flat, idx, N):
    B = idx.shape[0]
    off = (jnp.arange(B, dtype=jnp.int32).reshape((B,) + (1,) * (idx.ndim - 1))
           * jnp.int32(N))
    flat = (jnp.minimum(idx, N - 1).astype(jnp.int32) + off).reshape(-1)
    rows = _sc_gather(table_flat, flat)
    return rows.reshape(idx.shape + (table_flat.shape[-1],))


def _square_distance(src, dst):
    d = -2.0 * jnp.matmul(src, dst.transpose(0, 2, 1))
    d = d + jnp.sum(src ** 2, -1)[:, :, None]
    d = d + jnp.sum(dst ** 2, -1)[:, None, :]
    return d


def _fps_xla(xyz_t, npoint):
    xyz_t = jax.lax.stop_gradient(xyz_t)
    B, N, _ = xyz_t.shape
    def body(carry, _):
        distance, farthest = carry
        centroid = xyz_t[jnp.arange(B), farthest][:, None, :]
        dist = jnp.sum((xyz_t - centroid) ** 2, -1)
        distance = jnp.minimum(distance, dist)
        new_far = jnp.argmax(distance, -1).astype(jnp.int32)
        return (distance, new_far), farthest
    init = (jnp.full((B, N), 1e10, jnp.float32), jnp.zeros((B,), jnp.int32))
    _, idx = jax.lax.scan(body, init, None, length=npoint)
    return jnp.transpose(idx)


def _fps(xyz_t, npoint, interpret=False):
    B, N, _ = xyz_t.shape
    x = xyz_t[:, :, 0]
    y = xyz_t[:, :, 1]
    z = xyz_t[:, :, 2]

    def body(x_ref, y_ref, z_ref, o_ref):
        xv = x_ref[...]
        yv = y_ref[...]
        zv = z_ref[...]
        iota_n = jax.lax.broadcasted_iota(jnp.int32, (B, N), 1)
        iota_p = jax.lax.broadcasted_iota(jnp.int32, (B, npoint), 1)

        dist0 = (xv - xv[:, :1]) ** 2 + (yv - yv[:, :1]) ** 2 + (zv - zv[:, :1]) ** 2
        f0 = jnp.min(jnp.where(dist0 < -1.0, iota_n, 0), axis=1,
                     keepdims=True).astype(jnp.int32)
        o_ref[...] = jnp.where(iota_p >= 0, 0, iota_p)

        def step(i, carry):
            distance, f = carry
            o_ref[...] = jnp.where(iota_p == i, f, o_ref[...])
            m = iota_n == f
            cx = jnp.sum(jnp.where(m, xv, 0.0), axis=1, keepdims=True)
            cy = jnp.sum(jnp.where(m, yv, 0.0), axis=1, keepdims=True)
            cz = jnp.sum(jnp.where(m, zv, 0.0), axis=1, keepdims=True)
            dist = (xv - cx) ** 2 + (yv - cy) ** 2 + (zv - cz) ** 2
            distance = jnp.minimum(distance, dist)
            mx = jnp.max(distance, axis=1, keepdims=True)
            f = jnp.min(jnp.where(distance == mx, iota_n, N), axis=1,
                        keepdims=True).astype(jnp.int32)
            return distance, f

        jax.lax.fori_loop(0, npoint, step, (dist0, f0))

    return pl.pallas_call(
        body,
        out_shape=jax.ShapeDtypeStruct((B, npoint), jnp.int32),
        interpret=interpret,
    )(x, y, z)


def _query_ball(radius, nsample, xyz, new_xyz):
    B, N, _ = xyz.shape
    S = new_xyz.shape[1]
    sqrdists = jax.lax.stop_gradient(_square_distance(new_xyz, xyz))
    group_idx = jnp.broadcast_to(jnp.arange(N, dtype=jnp.int32), (B, S, N))
    group_idx = jnp.where(sqrdists > radius ** 2, N, group_idx)
    group_idx = jnp.sort(group_idx, axis=-1)[:, :, :nsample]
    group_first = group_idx[:, :, :1]
    group_idx = jnp.where(group_idx == N, group_first, group_idx)
    return group_idx


def _mlp_apply(x, layers):
    for p in layers:
        sh = (1, -1) + (1,) * (x.ndim - 2)
        y = jnp.einsum('oi,bi...->bo...', p['w'], x) + p['b'].reshape(sh)
        axes = (0,) + tuple(range(2, y.ndim))
        mean = jnp.mean(y, axes, keepdims=True)
        var = jnp.var(y, axes, keepdims=True)
        y = (y - mean) / jnp.sqrt(var + 1e-5)
        y = y * p['g'].reshape(sh) + p['be'].reshape(sh)
        x = jax.nn.relu(y)
    return x


def _sa_msg(xyz, points, npoint, radius_list, nsample_list, branch_params):
    xyz_t = xyz.transpose(0, 2, 1)
    points_t = points.transpose(0, 2, 1)
    B, N, _ = xyz_t.shape
    C = points_t.shape[-1]
    D = ((C + 3 + 127) // 128) * 128
    table = _pad_channels(jnp.concatenate([points_t, xyz_t], -1), D)
    table_flat = table.reshape(B * N, D)
    fps_idx = _fps_xla(xyz_t, npoint)
    new_xyz = _gather_points(table_flat, fps_idx, N)[..., C:C + 3]
    outs = []
    for radius, K, layers in zip(radius_list, nsample_list, branch_params):
        gidx = _query_ball(radius, K, xyz_t, new_xyz)
        rows = _gather_points(table_flat, gidx, N)
        grouped_xyz = rows[..., C:C + 3] - new_xyz[:, :, None, :]
        grouped_points = jnp.concatenate([rows[..., :C], grouped_xyz], -1)
        g = _mlp_apply(grouped_points.transpose(0, 3, 2, 1), layers)
        outs.append(jnp.max(g, axis=2))
    return new_xyz.transpose(0, 2, 1), jnp.concatenate(outs, axis=1)


def _sa_all(xyz, points, layers):
    xyz_t = xyz.transpose(0, 2, 1)
    points_t = points.transpose(0, 2, 1)
    new_points = jnp.concatenate([xyz_t[:, None], points_t[:, None]], -1)
    g = _mlp_apply(new_points.transpose(0, 3, 2, 1), layers)
    return jnp.zeros((xyz.shape[0], 3, 1), xyz.dtype), jnp.max(g, axis=2)


def _fp(xyz1, xyz2, points1, points2, layers):
    xyz1_t = xyz1.transpose(0, 2, 1)
    xyz2_t = xyz2.transpose(0, 2, 1)
    points2_t = points2.transpose(0, 2, 1)
    B, N, _ = xyz1_t.shape
    S = xyz2_t.shape[1]
    if S == 1:
        interpolated = jnp.tile(points2_t, (1, N, 1))
    else:
        dists = _square_distance(xyz1_t, xyz2_t)
        neg, idx = jax.lax.top_k(-dists, 3)
        d3 = -neg
        recip = 1.0 / (d3 + 1e-8)
        weight = recip / jnp.sum(recip, 2, keepdims=True)
        C2 = points2_t.shape[-1]
        D2 = ((C2 + 127) // 128) * 128
        t2 = _pad_channels(points2_t, D2).reshape(B * S, D2)
        nbr = _gather_points(t2, idx, S)[..., :C2]
        interpolated = jnp.sum(nbr * weight[..., None], axis=2)
    if points1 is not None:
        new_points = jnp.concatenate([points1.transpose(0, 2, 1), interpolated], -1)
    else:
        new_points = interpolated
    return _mlp_apply(new_points.transpose(0, 2, 1), layers)


def kernel(xyz, cls_label, params):
    B, C, N = xyz.shape
    l0_xyz = xyz[:, :3, :]
    l1_xyz, l1_points = _sa_msg(l0_xyz, xyz, 512, [0.1, 0.2, 0.4], [32, 64, 128], params['sa1'])
    l2_xyz, l2_points = _sa_msg(l1_xyz, l1_points, 128, [0.4, 0.8], [64, 128], params['sa2'])
    l3_xyz, l3_points = _sa_all(l2_xyz, l2_points, params['sa3'])
    l2_up = _fp(l2_xyz, l3_xyz, l2_points, l3_points, params['fp3'])
    l1_up = _fp(l1_xyz, l2_xyz, l1_points, l2_up, params['fp2'])
    cls_oh = jnp.tile(cls_label.reshape(B, 16, 1), (1, 1, N))
    l0_points = _fp(l0_xyz, l1_xyz, jnp.concatenate([cls_oh, l0_xyz, xyz], axis=1), l1_up, params['fp1'])
    y = jnp.einsum('oi,bin->bon', params['conv1']['w'], l0_points) + params['conv1']['b'][None, :, None]
    mean = jnp.mean(y, (0, 2), keepdims=True)
    var = jnp.var(y, (0, 2), keepdims=True)
    y = (y - mean) / jnp.sqrt(var + 1e-5)
    y = y * params['bn1']['g'][None, :, None] + params['bn1']['be'][None, :, None]
    y = jax.nn.relu(y)
    y = jnp.einsum('oi,bin->bon', params['conv2']['w'], y) + params['conv2']['b'][None, :, None]
    y = jax.nn.log_softmax(y, axis=1)
    return (y.transpose(0, 2, 1), l3_points)
